# baseline (speedup 1.0000x reference)
import functools

import jax
import jax.numpy as jnp
from jax import lax
from jax.experimental import pallas as pl
from jax.experimental.pallas import tpu as pltpu
from jax.experimental.pallas import tpu_sc as plsc

_NC = 2
_NS = 16
_NW = _NC * _NS
_LANES = 16

_SEG = 100
_SEGS_PER_CHUNK = 4


def _embed_body(seq, embed, n_flat, idx_hbm, table_hbm, pos_hbm, out_hbm,
                idx_v, rows_v, pos_v, gsem):
    chunk = _SEG * _SEGS_PER_CHUNK
    rows_per_chunk = chunk // seq
    per_worker = n_flat // _NW
    n_chunks = per_worker // chunk

    wid = lax.axis_index("s") * _NC + lax.axis_index("c")
    seg_base = wid * (per_worker // _SEG)
    tok_base = wid * per_worker

    pltpu.sync_copy(pos_hbm.at[pl.ds(0, seq)], pos_v)

    def chunk_body(g, carry):
        seg0 = seg_base + g * _SEGS_PER_CHUNK
        pltpu.sync_copy(idx_hbm.at[pl.ds(seg0, _SEGS_PER_CHUNK)], idx_v)
        copies = [
            pltpu.async_copy(
                table_hbm.at[idx_v.at[j]],
                rows_v.at[pl.ds(j * _SEG, _SEG)],
                gsem,
            )
            for j in range(_SEGS_PER_CHUNK)
        ]
        for cp in copies:
            cp.wait()

        def add_body(r, c2):
            for c in range(embed // _LANES):
                p = pos_v[r, pl.ds(c * _LANES, _LANES)]
                for rep in range(rows_per_chunk):
                    plsc.addupdate(
                        rows_v.at[rep * seq + r, pl.ds(c * _LANES, _LANES)], p)
            return c2

        lax.fori_loop(0, seq, add_body, 0)
        pltpu.sync_copy(rows_v, out_hbm.at[pl.ds(tok_base + g * chunk, chunk)])
        return carry

    lax.fori_loop(0, n_chunks, chunk_body, 0)


def kernel(token_ids, text_table, pos_table):
    batch, seq = token_ids.shape
    vocab, embed = text_table.shape
    n_flat = batch * seq
    chunk = _SEG * _SEGS_PER_CHUNK

    idx_flat = jnp.reshape(token_ids.astype(jnp.int32), (n_flat // _SEG, _SEG))

    mesh = plsc.VectorSubcoreMesh(core_axis_name="c", subcore_axis_name="s")
    body = functools.partial(_embed_body, seq, embed, n_flat)
    out = pl.kernel(
        body,
        out_type=jax.ShapeDtypeStruct((n_flat, embed), jnp.float32),
        mesh=mesh,
        scratch_types=[
            pltpu.VMEM((_SEGS_PER_CHUNK, _SEG), jnp.int32),
            pltpu.VMEM((chunk, embed), jnp.float32),
            pltpu.VMEM((seq, embed), jnp.float32),
            pltpu.SemaphoreType.DMA,
        ],
        compiler_params=pltpu.CompilerParams(use_tc_tiling_on_sc=False),
        name="sc_embed_lookup",
    )(idx_flat, text_table, pos_table)
    return jnp.reshape(out, (batch, seq, embed))
